# baseline (device time: 97648 ns/iter reference)
import jax
import jax.numpy as jnp
from jax import lax
from jax.experimental import pallas as pl
from jax.experimental.pallas import tpu as pltpu

N_DEV = 4
B, SQ, D = 2, 256, 512
H_LOCAL, DH = 4, 64
ROWS = B * SQ
EPS = 1e-5


def kernel(x, Wq, Wk, Wv, Wo, t_emb, W_mod, W_ff1, W_ff2):
    def body(
        x_ref, wq_ref, wk_ref, wv_ref, wo_ref, temb_ref, wmod_ref,
        wff1_ref, wff2_ref, out_ref, comm_ref, send_sems, recv_sems,
    ):
        my = lax.axis_index("i")
        right = lax.rem(my + 1, N_DEV)
        left = lax.rem(my + N_DEV - 1, N_DEV)

        barrier_sem = pltpu.get_barrier_semaphore()
        for nbr in (left, right):
            pl.semaphore_signal(
                barrier_sem, inc=1,
                device_id=(nbr,), device_id_type=pl.DeviceIdType.MESH,
            )
        pl.semaphore_wait(barrier_sem, 2)

        mod = []
        for b in range(B):
            mb = jnp.dot(
                temb_ref[b : b + 1, :], wmod_ref[...],
                preferred_element_type=jnp.float32,
            )
            mod.append([mb[:, i * D : (i + 1) * D] for i in range(6)])

        def ln_mod(h, scale, shift):
            m = jnp.mean(h, axis=-1, keepdims=True)
            v = jnp.mean((h - m) * (h - m), axis=-1, keepdims=True)
            return ((h - m) * lax.rsqrt(v + EPS)) * (1.0 + scale) + shift

        for b in range(B):
            sa, sha = mod[b][0], mod[b][1]
            xm = ln_mod(x_ref[b], sa, sha)
            q = jnp.dot(xm, wq_ref[...], preferred_element_type=jnp.float32)
            k = jnp.dot(xm, wk_ref[...], preferred_element_type=jnp.float32)
            v = jnp.dot(xm, wv_ref[...], preferred_element_type=jnp.float32)
            outs = []
            for h in range(H_LOCAL):
                sl = slice(h * DH, (h + 1) * DH)
                s = lax.dot_general(
                    q[:, sl], k[:, sl], (((1,), (1,)), ((), ())),
                    preferred_element_type=jnp.float32,
                ) * 0.125
                mx = jnp.max(s, axis=-1, keepdims=True)
                p = jnp.exp(s - mx)
                l = jnp.sum(p, axis=-1, keepdims=True)
                outs.append(
                    jnp.dot(p, v[:, sl], preferred_element_type=jnp.float32) / l
                )
            o = jnp.concatenate(outs, axis=1)
            comm_ref[0, b * SQ : (b + 1) * SQ, :] = jnp.dot(
                o, wo_ref[...], preferred_element_type=jnp.float32
            )

        def ring_allreduce(ar):
            base = ar * N_DEV
            acc = comm_ref[base]
            for h in range(N_DEV - 1):
                rdma = pltpu.make_async_remote_copy(
                    src_ref=comm_ref.at[base + h],
                    dst_ref=comm_ref.at[base + h + 1],
                    send_sem=send_sems.at[ar * 3 + h],
                    recv_sem=recv_sems.at[ar * 3 + h],
                    device_id=(right,),
                    device_id_type=pl.DeviceIdType.MESH,
                )
                rdma.start()
                rdma.wait()
                acc = acc + comm_ref[base + h + 1]
            return acc

        attn_full = ring_allreduce(0)

        x1 = []
        for b in range(B):
            ga = mod[b][2]
            x1b = x_ref[b] + ga * attn_full[b * SQ : (b + 1) * SQ, :]
            x1.append(x1b)
            sm, shm = mod[b][3], mod[b][4]
            xm2 = ln_mod(x1b, sm, shm)
            hb = jnp.dot(xm2, wff1_ref[...], preferred_element_type=jnp.float32)
            hb = hb / (1.0 + jnp.exp(-hb))
            comm_ref[N_DEV, b * SQ : (b + 1) * SQ, :] = jnp.dot(
                hb, wff2_ref[...], preferred_element_type=jnp.float32
            )

        ffn_full = ring_allreduce(1)

        for b in range(B):
            gm = mod[b][5]
            out_ref[b, :, :] = x1[b] + gm * ffn_full[b * SQ : (b + 1) * SQ, :]

    return pl.pallas_call(
        body,
        out_shape=jax.ShapeDtypeStruct((B, SQ, D), jnp.float32),
        in_specs=[pl.BlockSpec(memory_space=pltpu.VMEM)] * 9,
        out_specs=pl.BlockSpec(memory_space=pltpu.VMEM),
        scratch_shapes=[
            pltpu.VMEM((2 * N_DEV, ROWS, D), jnp.float32),
            pltpu.SemaphoreType.DMA((6,)),
            pltpu.SemaphoreType.DMA((6,)),
        ],
        compiler_params=pltpu.CompilerParams(collective_id=0),
    )(x, Wq, Wk, Wv, Wo, t_emb, W_mod, W_ff1, W_ff2)


# device time: 39706 ns/iter; 2.4593x vs baseline; 2.4593x over previous
import jax
import jax.numpy as jnp
from jax import lax
from jax.experimental import pallas as pl
from jax.experimental.pallas import tpu as pltpu

N_DEV = 4
B, SQ, D = 2, 256, 512
H_LOCAL, DH = 4, 64
ROWS = B * SQ
Q4 = ROWS // N_DEV
EPS = 1e-5
F32 = jnp.float32
BF16 = jnp.bfloat16


def kernel(x, Wq, Wk, Wv, Wo, t_emb, W_mod, W_ff1, W_ff2):
    def body(
        x_ref, wq_ref, wk_ref, wv_ref, wo_ref, temb_ref, wmod_ref,
        wff1_ref, wff2_ref, out_ref,
        pf_ref,
        x1_ref,
        p1_ref, p2_ref,
        rs1_ref, rs2_ref,
        red1_ref, red2_ref,
        send_sems, recv_sems,
    ):
        my = lax.axis_index("i")

        barrier_sem = pltpu.get_barrier_semaphore()
        for off in (1, 2, 3):
            pl.semaphore_signal(
                barrier_sem, inc=1,
                device_id=(lax.rem(my + off, N_DEV),),
                device_id_type=pl.DeviceIdType.MESH,
            )
        pl.semaphore_wait(barrier_sem, N_DEV - 1)

        def exchange(phase, make_src, make_dst):
            rdmas = []
            for off in (1, 2, 3):
                dev = lax.rem(my + off, N_DEV)
                rdma = pltpu.make_async_remote_copy(
                    src_ref=make_src(off, dev),
                    dst_ref=make_dst(off, dev),
                    send_sem=send_sems.at[phase * 3 + off - 1],
                    recv_sem=recv_sems.at[phase * 3 + off - 1],
                    device_id=(dev,),
                    device_id_type=pl.DeviceIdType.MESH,
                )
                rdma.start()
                rdmas.append(rdma)
            return rdmas

        def store_quarters(ref, val):
            for q in range(N_DEV):
                ref[q] = val[q * Q4 : (q + 1) * Q4, :].astype(ref.dtype)

        def all_reduce(phase0, p_ref, rs_ref):
            rdmas = exchange(
                phase0,
                lambda off, dev: p_ref.at[dev],
                lambda off, dev: rs_ref.at[off - 1],
            )
            for r in rdmas:
                r.wait()
            total = pf_ref[my]
            for i in range(N_DEV - 1):
                total = total + rs_ref[i].astype(F32)
            return total

        def all_gather(phase0, red_ref):
            rdmas = exchange(
                phase0,
                lambda off, dev: red_ref.at[my],
                lambda off, dev: red_ref.at[my],
            )
            for r in rdmas:
                r.wait()

        mod = []
        for b in range(B):
            mb = jnp.dot(
                temb_ref[b : b + 1, :], wmod_ref[...],
                preferred_element_type=F32,
            )
            mod.append([mb[:, i * D : (i + 1) * D] for i in range(6)])

        def ln_mod(h, scale, shift):
            m = jnp.mean(h, axis=-1, keepdims=True)
            v = jnp.mean((h - m) * (h - m), axis=-1, keepdims=True)
            return ((h - m) * lax.rsqrt(v + EPS)) * (1.0 + scale) + shift

        attn_parts = []
        for b in range(B):
            sa, sha = mod[b][0], mod[b][1]
            xm = ln_mod(x_ref[b], sa, sha)
            q = jnp.dot(xm, wq_ref[...], preferred_element_type=F32)
            k = jnp.dot(xm, wk_ref[...], preferred_element_type=F32)
            v = jnp.dot(xm, wv_ref[...], preferred_element_type=F32)
            outs = []
            for h in range(H_LOCAL):
                sl = slice(h * DH, (h + 1) * DH)
                s = lax.dot_general(
                    q[:, sl], k[:, sl], (((1,), (1,)), ((), ())),
                    preferred_element_type=F32,
                ) * 0.125
                mx = jnp.max(s, axis=-1, keepdims=True)
                p = jnp.exp(s - mx)
                l = jnp.sum(p, axis=-1, keepdims=True)
                outs.append(
                    jnp.dot(p, v[:, sl], preferred_element_type=F32) / l
                )
            o = jnp.concatenate(outs, axis=1)
            attn_parts.append(
                jnp.dot(o, wo_ref[...], preferred_element_type=F32)
            )
        attn_partial = jnp.concatenate(attn_parts, axis=0)
        store_quarters(pf_ref, attn_partial)
        store_quarters(p1_ref, attn_partial)

        red_my = all_reduce(0, p1_ref, rs1_ref)
        red1_ref[my] = red_my.astype(BF16)
        all_gather(1, red1_ref)
        attn_full = jnp.concatenate(
            [red1_ref[q].astype(F32) for q in range(N_DEV)], axis=0
        )

        x1_parts = []
        for b in range(B):
            ga = mod[b][2]
            x1_parts.append(
                x_ref[b] + ga * attn_full[b * SQ : (b + 1) * SQ, :]
            )
        x1 = jnp.concatenate(x1_parts, axis=0)
        store_quarters(x1_ref, x1)

        ffn_parts = []
        for b in range(B):
            sm, shm = mod[b][3], mod[b][4]
            xm2 = ln_mod(x1[b * SQ : (b + 1) * SQ, :], sm, shm)
            hb = jnp.dot(xm2, wff1_ref[...], preferred_element_type=F32)
            hb = hb / (1.0 + jnp.exp(-hb))
            ffn_parts.append(
                jnp.dot(hb, wff2_ref[...], preferred_element_type=F32)
            )
        ffn_partial = jnp.concatenate(ffn_parts, axis=0)
        store_quarters(pf_ref, ffn_partial)
        store_quarters(p2_ref, ffn_partial)

        red2_my = all_reduce(2, p2_ref, rs2_ref)
        gm_my = jnp.where(my >= 2, mod[1][5], mod[0][5])
        out_my = x1_ref[my] + gm_my * red2_my
        red2_ref[my] = out_my.astype(BF16)
        all_gather(3, red2_ref)

        for b in range(B):
            out_ref[b, :, :] = jnp.concatenate(
                [red2_ref[2 * b].astype(F32), red2_ref[2 * b + 1].astype(F32)],
                axis=0,
            )

    return pl.pallas_call(
        body,
        out_shape=jax.ShapeDtypeStruct((B, SQ, D), jnp.float32),
        in_specs=[pl.BlockSpec(memory_space=pltpu.VMEM)] * 9,
        out_specs=pl.BlockSpec(memory_space=pltpu.VMEM),
        scratch_shapes=[
            pltpu.VMEM((N_DEV, Q4, D), F32),
            pltpu.VMEM((N_DEV, Q4, D), F32),
            pltpu.VMEM((N_DEV, Q4, D), BF16),
            pltpu.VMEM((N_DEV, Q4, D), BF16),
            pltpu.VMEM((3, Q4, D), BF16),
            pltpu.VMEM((3, Q4, D), BF16),
            pltpu.VMEM((N_DEV, Q4, D), BF16),
            pltpu.VMEM((N_DEV, Q4, D), BF16),
            pltpu.SemaphoreType.DMA((12,)),
            pltpu.SemaphoreType.DMA((12,)),
        ],
        compiler_params=pltpu.CompilerParams(collective_id=0),
    )(x, Wq, Wk, Wv, Wo, t_emb, W_mod, W_ff1, W_ff2)


# device time: 39424 ns/iter; 2.4769x vs baseline; 1.0072x over previous
import jax
import jax.numpy as jnp
from jax import lax
from jax.experimental import pallas as pl
from jax.experimental.pallas import tpu as pltpu

N_DEV = 4
B, SQ, D = 2, 256, 512
H_LOCAL, DH = 4, 64
ROWS = B * SQ
Q4 = ROWS // N_DEV
EPS = 1e-5
F32 = jnp.float32
BF16 = jnp.bfloat16


def kernel(x, Wq, Wk, Wv, Wo, t_emb, W_mod, W_ff1, W_ff2):
    def body(
        x_ref, wq_ref, wk_ref, wv_ref, wo_ref, temb_ref, wmod_ref,
        wff1_ref, wff2_ref, out_ref,
        pf_ref,
        x1_ref,
        p1_ref, p2_ref,
        rs1_ref, rs2_ref,
        red1_ref, red2_ref,
        send_sems, recv_sems,
    ):
        my = lax.axis_index("i")

        barrier_sem = pltpu.get_barrier_semaphore()
        for off in (1, 2, 3):
            pl.semaphore_signal(
                barrier_sem, inc=1,
                device_id=(lax.rem(my + off, N_DEV),),
                device_id_type=pl.DeviceIdType.MESH,
            )
        pl.semaphore_wait(barrier_sem, N_DEV - 1)

        def exchange(phase, make_src, make_dst):
            rdmas = []
            for off in (1, 2, 3):
                dev = lax.rem(my + off, N_DEV)
                rdma = pltpu.make_async_remote_copy(
                    src_ref=make_src(off, dev),
                    dst_ref=make_dst(off, dev),
                    send_sem=send_sems.at[phase * 3 + off - 1],
                    recv_sem=recv_sems.at[phase * 3 + off - 1],
                    device_id=(dev,),
                    device_id_type=pl.DeviceIdType.MESH,
                )
                rdma.start()
                rdmas.append(rdma)
            return rdmas

        def store_quarters(ref, val):
            for q in range(N_DEV):
                ref[q] = val[q * Q4 : (q + 1) * Q4, :].astype(ref.dtype)

        def all_reduce(phase0, p_ref, rs_ref):
            rdmas = exchange(
                phase0,
                lambda off, dev: p_ref.at[dev],
                lambda off, dev: rs_ref.at[off - 1],
            )
            for r in rdmas:
                r.wait()
            total = pf_ref[my]
            for i in range(N_DEV - 1):
                total = total + rs_ref[i].astype(F32)
            return total

        def all_gather(phase0, red_ref):
            rdmas = exchange(
                phase0,
                lambda off, dev: red_ref.at[my],
                lambda off, dev: red_ref.at[my],
            )
            for r in rdmas:
                r.wait()

        mod = []
        for b in range(B):
            mb = jnp.dot(
                temb_ref[b : b + 1, :], wmod_ref[...],
                preferred_element_type=F32,
            )
            mod.append([mb[:, i * D : (i + 1) * D] for i in range(6)])

        def ln_mod(h, scale, shift):
            m = jnp.mean(h, axis=-1, keepdims=True)
            v = jnp.mean((h - m) * (h - m), axis=-1, keepdims=True)
            return ((h - m) * lax.rsqrt(v + EPS)) * (1.0 + scale) + shift

        wq_b = wq_ref[...].astype(BF16)
        wk_b = wk_ref[...].astype(BF16)
        wv_b = wv_ref[...].astype(BF16)
        wo_b = wo_ref[...].astype(BF16)
        attn_parts = []
        for b in range(B):
            sa, sha = mod[b][0], mod[b][1]
            xm = ln_mod(x_ref[b], sa, sha).astype(BF16)
            q = jnp.dot(xm, wq_b, preferred_element_type=F32).astype(BF16)
            k = jnp.dot(xm, wk_b, preferred_element_type=F32).astype(BF16)
            v = jnp.dot(xm, wv_b, preferred_element_type=F32).astype(BF16)
            outs = []
            for h in range(H_LOCAL):
                sl = slice(h * DH, (h + 1) * DH)
                s = lax.dot_general(
                    q[:, sl], k[:, sl], (((1,), (1,)), ((), ())),
                    preferred_element_type=F32,
                ) * 0.125
                mx = jnp.max(s, axis=-1, keepdims=True)
                p = jnp.exp(s - mx)
                l = jnp.sum(p, axis=-1, keepdims=True)
                outs.append(
                    jnp.dot(
                        p.astype(BF16), v[:, sl], preferred_element_type=F32
                    ) / l
                )
            o = jnp.concatenate(outs, axis=1).astype(BF16)
            attn_parts.append(
                jnp.dot(o, wo_b, preferred_element_type=F32)
            )
        attn_partial = jnp.concatenate(attn_parts, axis=0)
        store_quarters(pf_ref, attn_partial)
        store_quarters(p1_ref, attn_partial)

        red_my = all_reduce(0, p1_ref, rs1_ref)
        red1_ref[my] = red_my.astype(BF16)
        all_gather(1, red1_ref)
        attn_full = jnp.concatenate(
            [red1_ref[q].astype(F32) for q in range(N_DEV)], axis=0
        )

        x1_parts = []
        for b in range(B):
            ga = mod[b][2]
            x1_parts.append(
                x_ref[b] + ga * attn_full[b * SQ : (b + 1) * SQ, :]
            )
        x1 = jnp.concatenate(x1_parts, axis=0)
        store_quarters(x1_ref, x1)

        wff1_b = wff1_ref[...].astype(BF16)
        wff2_b = wff2_ref[...].astype(BF16)
        ffn_parts = []
        for b in range(B):
            sm, shm = mod[b][3], mod[b][4]
            xm2 = ln_mod(x1[b * SQ : (b + 1) * SQ, :], sm, shm).astype(BF16)
            hb = jnp.dot(xm2, wff1_b, preferred_element_type=F32)
            hb = hb / (1.0 + jnp.exp(-hb))
            ffn_parts.append(
                jnp.dot(hb.astype(BF16), wff2_b, preferred_element_type=F32)
            )
        ffn_partial = jnp.concatenate(ffn_parts, axis=0)
        store_quarters(pf_ref, ffn_partial)
        store_quarters(p2_ref, ffn_partial)

        red2_my = all_reduce(2, p2_ref, rs2_ref)
        gm_my = jnp.where(my >= 2, mod[1][5], mod[0][5])
        out_my = x1_ref[my] + gm_my * red2_my
        red2_ref[my] = out_my.astype(BF16)
        all_gather(3, red2_ref)

        for b in range(B):
            out_ref[b, :, :] = jnp.concatenate(
                [red2_ref[2 * b].astype(F32), red2_ref[2 * b + 1].astype(F32)],
                axis=0,
            )

    return pl.pallas_call(
        body,
        out_shape=jax.ShapeDtypeStruct((B, SQ, D), jnp.float32),
        in_specs=[pl.BlockSpec(memory_space=pltpu.VMEM)] * 9,
        out_specs=pl.BlockSpec(memory_space=pltpu.VMEM),
        scratch_shapes=[
            pltpu.VMEM((N_DEV, Q4, D), F32),
            pltpu.VMEM((N_DEV, Q4, D), F32),
            pltpu.VMEM((N_DEV, Q4, D), BF16),
            pltpu.VMEM((N_DEV, Q4, D), BF16),
            pltpu.VMEM((3, Q4, D), BF16),
            pltpu.VMEM((3, Q4, D), BF16),
            pltpu.VMEM((N_DEV, Q4, D), BF16),
            pltpu.VMEM((N_DEV, Q4, D), BF16),
            pltpu.SemaphoreType.DMA((12,)),
            pltpu.SemaphoreType.DMA((12,)),
        ],
        compiler_params=pltpu.CompilerParams(collective_id=0),
    )(x, Wq, Wk, Wv, Wo, t_emb, W_mod, W_ff1, W_ff2)


# device time: 38034 ns/iter; 2.5674x vs baseline; 1.0365x over previous
import jax
import jax.numpy as jnp
from jax import lax
from jax.experimental import pallas as pl
from jax.experimental.pallas import tpu as pltpu

N_DEV = 4
B, SQ, D = 2, 256, 512
H_LOCAL, DH = 4, 64
ROWS = B * SQ
Q4 = ROWS // N_DEV
EPS = 1e-5
F32 = jnp.float32
BF16 = jnp.bfloat16


def kernel(x, Wq, Wk, Wv, Wo, t_emb, W_mod, W_ff1, W_ff2):
    def body(
        x_ref, wq_ref, wk_ref, wv_ref, wo_ref, temb_ref, wmod_ref,
        wff1_ref, wff2_ref, out_ref,
        x0q_ref,
        pf_ref,
        p1_ref,
        rs1_ref,
        x1ag_ref,
        p2_ref,
        rs2_ref,
        red2_ref,
        send_sems, recv_sems,
    ):
        my = lax.axis_index("i")

        barrier_sem = pltpu.get_barrier_semaphore()
        for off in (1, 2, 3):
            pl.semaphore_signal(
                barrier_sem, inc=1,
                device_id=(lax.rem(my + off, N_DEV),),
                device_id_type=pl.DeviceIdType.MESH,
            )
        pl.semaphore_wait(barrier_sem, N_DEV - 1)

        def mk_rdma(phase, off, src, dst, dev):
            return pltpu.make_async_remote_copy(
                src_ref=src,
                dst_ref=dst,
                send_sem=send_sems.at[phase * 3 + off - 1],
                recv_sem=recv_sems.at[phase * 3 + off - 1],
                device_id=(dev,),
                device_id_type=pl.DeviceIdType.MESH,
            )

        def exchange(phase, make_src, make_dst):
            rdmas = []
            for off in (1, 2, 3):
                dev = lax.rem(my + off, N_DEV)
                rdma = mk_rdma(phase, off, make_src(off, dev),
                               make_dst(off, dev), dev)
                rdma.start()
                rdmas.append(rdma)
            return rdmas

        def store_quarters(ref, val):
            for q in range(N_DEV):
                ref[q] = val[q * Q4 : (q + 1) * Q4, :].astype(ref.dtype)

        def bsel(pair, qidx):
            return jnp.where(qidx >= 2, pair[1], pair[0])

        mod = []
        for b in range(B):
            mb = jnp.dot(
                temb_ref[b : b + 1, :], wmod_ref[...],
                preferred_element_type=F32,
            )
            mod.append([mb[:, i * D : (i + 1) * D] for i in range(6)])
        sa_, sha_, ga_, sm_, shm_, gm_ = (
            [mod[0][i], mod[1][i]] for i in range(6)
        )

        for q in range(N_DEV):
            x0q_ref[q] = x_ref[q // 2][(q % 2) * Q4 : (q % 2 + 1) * Q4, :]

        def ln_mod(h, scale, shift):
            m = jnp.mean(h, axis=-1, keepdims=True)
            v = jnp.mean((h - m) * (h - m), axis=-1, keepdims=True)
            return ((h - m) * lax.rsqrt(v + EPS)) * (1.0 + scale) + shift

        wq_b = wq_ref[...].astype(BF16)
        wk_b = wk_ref[...].astype(BF16)
        wv_b = wv_ref[...].astype(BF16)
        wo_b = wo_ref[...].astype(BF16)
        attn_parts = []
        for b in range(B):
            xm = ln_mod(x_ref[b], sa_[b], sha_[b]).astype(BF16)
            q = jnp.dot(xm, wq_b, preferred_element_type=F32).astype(BF16)
            k = jnp.dot(xm, wk_b, preferred_element_type=F32).astype(BF16)
            v = jnp.dot(xm, wv_b, preferred_element_type=F32).astype(BF16)
            outs = []
            for h in range(H_LOCAL):
                sl = slice(h * DH, (h + 1) * DH)
                s = lax.dot_general(
                    q[:, sl], k[:, sl], (((1,), (1,)), ((), ())),
                    preferred_element_type=F32,
                ) * 0.125
                mx = jnp.max(s, axis=-1, keepdims=True)
                p = jnp.exp(s - mx)
                l = jnp.sum(p, axis=-1, keepdims=True)
                outs.append(
                    jnp.dot(
                        p.astype(BF16), v[:, sl], preferred_element_type=F32
                    ) / l
                )
            o = jnp.concatenate(outs, axis=1).astype(BF16)
            attn_parts.append(jnp.dot(o, wo_b, preferred_element_type=F32))
        attn_partial = jnp.concatenate(attn_parts, axis=0)
        store_quarters(pf_ref, attn_partial)
        store_quarters(p1_ref, attn_partial)

        rs1 = exchange(
            0,
            lambda off, dev: p1_ref.at[dev],
            lambda off, dev: rs1_ref.at[off - 1],
        )
        for r in rs1:
            r.wait()
        attn_my = pf_ref[my]
        for i in range(N_DEV - 1):
            attn_my = attn_my + rs1_ref[i].astype(F32)

        x1_my = x0q_ref[my] + bsel(ga_, my) * attn_my
        x1ag_ref[my] = x1_my.astype(BF16)
        ag1 = exchange(
            1,
            lambda off, dev: x1ag_ref.at[my],
            lambda off, dev: x1ag_ref.at[my],
        )

        wff1_b = wff1_ref[...].astype(BF16)
        wff2_b = wff2_ref[...].astype(BF16)

        def ffn_block(x1_blk, qidx):
            xm2 = ln_mod(x1_blk, bsel(sm_, qidx), bsel(shm_, qidx))
            hb = jnp.dot(xm2.astype(BF16), wff1_b, preferred_element_type=F32)
            hb = hb / (1.0 + jnp.exp(-hb))
            return jnp.dot(hb.astype(BF16), wff2_b, preferred_element_type=F32)

        ffn_my = ffn_block(x1_my, my)

        rs2 = []
        for off in (1, 2, 3):
            ag1[off - 1].wait_recv()
            qidx = lax.rem(my - off + N_DEV, N_DEV)
            fblk = ffn_block(x1ag_ref[qidx].astype(F32), qidx)
            p2_ref[qidx] = fblk.astype(BF16)
            off_send = N_DEV - off
            r = mk_rdma(
                2, off_send, p2_ref.at[qidx], rs2_ref.at[off_send - 1], qidx
            )
            r.start()
            rs2.append(r)

        for r in rs2:
            r.wait_recv()
        total2 = ffn_my
        for i in range(N_DEV - 1):
            total2 = total2 + rs2_ref[i].astype(F32)
        out_my = x1_my + bsel(gm_, my) * total2
        red2_ref[my] = out_my.astype(BF16)
        ag2 = exchange(
            3,
            lambda off, dev: red2_ref.at[my],
            lambda off, dev: red2_ref.at[my],
        )
        for r in ag2:
            r.wait_recv()

        for b in range(B):
            out_ref[b, :, :] = jnp.concatenate(
                [red2_ref[2 * b].astype(F32), red2_ref[2 * b + 1].astype(F32)],
                axis=0,
            )

        for r in ag1:
            r.wait_send()
        for r in rs2:
            r.wait_send()
        for r in ag2:
            r.wait_send()

    return pl.pallas_call(
        body,
        out_shape=jax.ShapeDtypeStruct((B, SQ, D), jnp.float32),
        in_specs=[pl.BlockSpec(memory_space=pltpu.VMEM)] * 9,
        out_specs=pl.BlockSpec(memory_space=pltpu.VMEM),
        scratch_shapes=[
            pltpu.VMEM((N_DEV, Q4, D), F32),
            pltpu.VMEM((N_DEV, Q4, D), F32),
            pltpu.VMEM((N_DEV, Q4, D), BF16),
            pltpu.VMEM((3, Q4, D), BF16),
            pltpu.VMEM((N_DEV, Q4, D), BF16),
            pltpu.VMEM((N_DEV, Q4, D), BF16),
            pltpu.VMEM((3, Q4, D), BF16),
            pltpu.VMEM((N_DEV, Q4, D), BF16),
            pltpu.SemaphoreType.DMA((12,)),
            pltpu.SemaphoreType.DMA((12,)),
        ],
        compiler_params=pltpu.CompilerParams(collective_id=0),
    )(x, Wq, Wk, Wv, Wo, t_emb, W_mod, W_ff1, W_ff2)
